# Initial kernel scaffold; baseline (speedup 1.0000x reference)
import functools

import jax
import jax.numpy as jnp
from jax import lax
from jax.experimental import pallas as pl
from jax.experimental.pallas import tpu as pltpu

D_MODEL = 2048
N_HEADS = 16
HEAD_DIM = D_MODEL // N_HEADS
IDX_HEADS = 4
IDX_DIM = 64
S = 2048
NEG_INF = -1e9

BQ = 256
N_QBLK = S // BQ
WBLK = 512

_HI = jax.lax.Precision.HIGHEST


def _dot(a, b, dims, precision=_HI):
    return lax.dot_general(a, b, dimension_numbers=(dims, ((), ())),
                           preferred_element_type=jnp.float32,
                           precision=precision)


def _proj_body(h_ref, w_ref, out_ref, pooled_ref):
    j = pl.program_id(0)
    h = h_ref[...]
    out_ref[...] = _dot(h, w_ref[...], ((1,), (0,)))

    @pl.when(j == 0)
    def _():
        pooled_ref[...] = jnp.sum(h, axis=0, keepdims=True) * (1.0 / S)


def _run_proj(h, w_cat):
    n_wblk = w_cat.shape[1] // WBLK
    out, pooled = pl.pallas_call(
        _proj_body,
        grid=(n_wblk,),
        in_specs=[
            pl.BlockSpec((S, D_MODEL), lambda j: (0, 0)),
            pl.BlockSpec((D_MODEL, WBLK), lambda j: (0, j)),
        ],
        out_specs=[
            pl.BlockSpec((S, WBLK), lambda j: (0, j)),
            pl.BlockSpec((1, D_MODEL), lambda j: (0, 0)),
        ],
        out_shape=[
            jax.ShapeDtypeStruct((S, w_cat.shape[1]), jnp.float32),
            jax.ShapeDtypeStruct((1, D_MODEL), jnp.float32),
        ],
    )(h, w_cat)
    return out, pooled


def _f32_key(x):
    u = lax.bitcast_convert_type(x, jnp.uint32)
    neg = (u >> 31) == jnp.uint32(1)
    return jnp.where(neg, ~u, u | jnp.uint32(0x80000000))


def _key_to_f32(key):
    neg = (key >> 31) == jnp.uint32(0)
    u = jnp.where(neg, ~key, key & jnp.uint32(0x7FFFFFFF))
    return lax.bitcast_convert_type(u, jnp.float32)


def _scores_body(qi_ref, ki_ref, iw_ref, pooled_ref, cw_ref, cb_ref,
                 scores_ref, thr_ref):
    acc = jnp.zeros((BQ, S), jnp.float32)
    for hh in range(IDX_HEADS):
        q = qi_ref[:, hh * IDX_DIM:(hh + 1) * IDX_DIM]
        k = ki_ref[:, hh * IDX_DIM:(hh + 1) * IDX_DIM]
        dp = _dot(q, k, ((1,), (1,)))
        acc = acc + iw_ref[hh] * jnp.maximum(dp, 0.0)
    scores_ref[...] = acc

    r = _dot(pooled_ref[...], cw_ref[...], ((1,), (0,)))[0, 0] + cb_ref[0]
    ratio = 1.0 / (1.0 + jnp.exp(-r))
    kf = jnp.clip(lax.round(ratio * S, lax.RoundingMethod.TO_NEAREST_EVEN),
                  1.0, float(S))
    kint = kf.astype(jnp.int32)

    keys = _f32_key(acc)

    def step(i, cur):
        bit = jnp.uint32(1) << (jnp.uint32(31) - i.astype(jnp.uint32))
        cand = cur | bit
        cnt = jnp.sum((keys >= cand).astype(jnp.int32), axis=1, keepdims=True)
        return jnp.where(cnt >= kint, cand, cur)

    cur = lax.fori_loop(0, 32, step, jnp.zeros((BQ, 1), jnp.uint32))
    thr_ref[...] = _key_to_f32(cur)


def _run_scores(qi, ki, iw, pooled, cw, cb):
    return pl.pallas_call(
        _scores_body,
        grid=(N_QBLK,),
        in_specs=[
            pl.BlockSpec((BQ, IDX_HEADS * IDX_DIM), lambda i: (i, 0)),
            pl.BlockSpec((S, IDX_HEADS * IDX_DIM), lambda i: (0, 0)),
            pl.BlockSpec(memory_space=pltpu.SMEM),
            pl.BlockSpec((1, D_MODEL), lambda i: (0, 0)),
            pl.BlockSpec((D_MODEL, 1), lambda i: (0, 0)),
            pl.BlockSpec(memory_space=pltpu.SMEM),
        ],
        out_specs=[
            pl.BlockSpec((BQ, S), lambda i: (i, 0)),
            pl.BlockSpec((BQ, 1), lambda i: (i, 0)),
        ],
        out_shape=[
            jax.ShapeDtypeStruct((S, S), jnp.float32),
            jax.ShapeDtypeStruct((S, 1), jnp.float32),
        ],
    )(qi, ki, iw, pooled, cw, cb)


def _attn_body(q_ref, k_ref, v_ref, scores_ref, thr_ref, out_ref):
    add_mask = jnp.where(scores_ref[...] >= thr_ref[...], 0.0, NEG_INF)
    scale = 1.0 / (HEAD_DIM ** 0.5)
    for hh in range(N_HEADS):
        sl = slice(hh * HEAD_DIM, (hh + 1) * HEAD_DIM)
        logits = _dot(q_ref[:, sl], k_ref[:, sl], ((1,), (1,))) * scale
        logits = logits + add_mask
        m = jnp.max(logits, axis=1, keepdims=True)
        p = jnp.exp(logits - m)
        p = p / jnp.sum(p, axis=1, keepdims=True)
        out_ref[:, sl] = _dot(p, v_ref[:, sl], ((1,), (0,)))


def _run_attn(q, k, v, scores, thr):
    return pl.pallas_call(
        _attn_body,
        grid=(N_QBLK,),
        in_specs=[
            pl.BlockSpec((BQ, D_MODEL), lambda i: (i, 0)),
            pl.BlockSpec((S, D_MODEL), lambda i: (0, 0)),
            pl.BlockSpec((S, D_MODEL), lambda i: (0, 0)),
            pl.BlockSpec((BQ, S), lambda i: (i, 0)),
            pl.BlockSpec((BQ, 1), lambda i: (i, 0)),
        ],
        out_specs=pl.BlockSpec((BQ, D_MODEL), lambda i: (i, 0)),
        out_shape=jax.ShapeDtypeStruct((S, D_MODEL), jnp.float32),
    )(q, k, v, scores, thr)


def _oproj_body(x_ref, w_ref, out_ref):
    out_ref[...] = _dot(x_ref[...], w_ref[...], ((1,), (0,)))


def _run_oproj(x, w):
    return pl.pallas_call(
        _oproj_body,
        grid=(N_QBLK,),
        in_specs=[
            pl.BlockSpec((BQ, D_MODEL), lambda i: (i, 0)),
            pl.BlockSpec((D_MODEL, D_MODEL), lambda i: (0, 0)),
        ],
        out_specs=pl.BlockSpec((BQ, D_MODEL), lambda i: (i, 0)),
        out_shape=jax.ShapeDtypeStruct((S, D_MODEL), jnp.float32),
    )(x, w)


def kernel(hidden_states, q_idx_W, q_idx_b, k_idx_W, k_idx_b, idx_weights,
           ctrl_W, ctrl_b, Wq, Wk, Wv, Wo):
    h = hidden_states[0]

    w_cat = jnp.concatenate([q_idx_W, k_idx_W, Wq, Wk, Wv], axis=1)
    proj, pooled = _run_proj(h, w_cat)

    nidx = IDX_HEADS * IDX_DIM
    qi = proj[:, :nidx] + q_idx_b[None, :]
    ki = proj[:, nidx:2 * nidx] + k_idx_b[None, :]
    q = proj[:, 2 * nidx:2 * nidx + D_MODEL]
    k = proj[:, 2 * nidx + D_MODEL:2 * nidx + 2 * D_MODEL]
    v = proj[:, 2 * nidx + 2 * D_MODEL:]

    scores, thr = _run_scores(qi, ki, idx_weights, pooled, ctrl_W, ctrl_b)

    attn = _run_attn(q, k, v, scores, thr)
    out = _run_oproj(attn, Wo)
    return out[None]


# speedup vs baseline: 2.7118x; 2.7118x over previous
import functools

import jax
import jax.numpy as jnp
from jax import lax
from jax.experimental import pallas as pl
from jax.experimental.pallas import tpu as pltpu

D_MODEL = 2048
N_HEADS = 16
HEAD_DIM = D_MODEL // N_HEADS
IDX_HEADS = 4
IDX_DIM = 64
S = 2048
NEG_INF = -1e9

BQ = 256
N_QBLK = S // BQ
WBLK = 256

def _dot(a, b, dims):
    return lax.dot_general(a.astype(jnp.bfloat16), b.astype(jnp.bfloat16),
                           dimension_numbers=(dims, ((), ())),
                           preferred_element_type=jnp.float32)


def _proj_body(h_ref, w_ref, out_ref, pooled_ref):
    i = pl.program_id(0)
    j = pl.program_id(1)
    out_ref[...] = _dot(h_ref[...], w_ref[...], ((1,), (0,)))

    @pl.when(j == 0)
    def _():
        part = jnp.sum(h_ref[...], axis=0, keepdims=True) * (1.0 / S)

        @pl.when(i == 0)
        def _():
            pooled_ref[...] = part

        @pl.when(i > 0)
        def _():
            pooled_ref[...] += part


def _run_proj(h, w_cat):
    bm = 512
    n_wblk = w_cat.shape[1] // WBLK
    return pl.pallas_call(
        _proj_body,
        grid=(S // bm, n_wblk),
        in_specs=[
            pl.BlockSpec((bm, D_MODEL), lambda i, j: (i, 0)),
            pl.BlockSpec((D_MODEL, WBLK), lambda i, j: (0, j)),
        ],
        out_specs=[
            pl.BlockSpec((bm, WBLK), lambda i, j: (i, j)),
            pl.BlockSpec((1, D_MODEL), lambda i, j: (0, 0)),
        ],
        out_shape=[
            jax.ShapeDtypeStruct((S, w_cat.shape[1]), jnp.float32),
            jax.ShapeDtypeStruct((1, D_MODEL), jnp.float32),
        ],
    )(h, w_cat)


def _f32_key(x):
    u = lax.bitcast_convert_type(x, jnp.uint32)
    neg = (u >> 31) == jnp.uint32(1)
    return jnp.where(neg, ~u, u | jnp.uint32(0x80000000))


def _key_to_f32(key):
    neg = (key >> 31) == jnp.uint32(0)
    u = jnp.where(neg, ~key, key & jnp.uint32(0x7FFFFFFF))
    return lax.bitcast_convert_type(u, jnp.float32)


def _scores_body(qi_ref, ki_ref, qb_ref, kb_ref, iw_ref, pooled_ref, cw_ref,
                 cb_ref, scores_ref, thr_ref):
    acc = jnp.zeros((BQ, S), jnp.float32)
    for hh in range(IDX_HEADS):
        sl = slice(hh * IDX_DIM, (hh + 1) * IDX_DIM)
        q = qi_ref[:, sl] + qb_ref[:, sl]
        k = ki_ref[:, sl] + kb_ref[:, sl]
        dp = _dot(q, k, ((1,), (1,)))
        acc = acc + iw_ref[hh] * jnp.maximum(dp, 0.0)
    scores_ref[...] = acc

    r = _dot(pooled_ref[...], cw_ref[...], ((1,), (0,)))[0, 0] + cb_ref[0]
    ratio = 1.0 / (1.0 + jnp.exp(-r))
    kf = jnp.clip(lax.round(ratio * S, lax.RoundingMethod.TO_NEAREST_EVEN),
                  1.0, float(S))
    kint = kf.astype(jnp.int32)

    keys = _f32_key(acc)

    def step(i, cur):
        bit = jnp.uint32(1) << (jnp.uint32(31) - i.astype(jnp.uint32))
        cand = cur | bit
        cnt = jnp.sum((keys >= cand).astype(jnp.int32), axis=1, keepdims=True)
        return jnp.where(cnt >= kint, cand, cur)

    cur = lax.fori_loop(0, 32, step, jnp.zeros((BQ, 1), jnp.uint32))
    thr_ref[...] = _key_to_f32(cur)


def _run_scores(qi, ki, qb, kb, iw, pooled, cw, cb):
    return pl.pallas_call(
        _scores_body,
        grid=(N_QBLK,),
        in_specs=[
            pl.BlockSpec((BQ, IDX_HEADS * IDX_DIM), lambda i: (i, 0)),
            pl.BlockSpec((S, IDX_HEADS * IDX_DIM), lambda i: (0, 0)),
            pl.BlockSpec((1, IDX_HEADS * IDX_DIM), lambda i: (0, 0)),
            pl.BlockSpec((1, IDX_HEADS * IDX_DIM), lambda i: (0, 0)),
            pl.BlockSpec(memory_space=pltpu.SMEM),
            pl.BlockSpec((1, D_MODEL), lambda i: (0, 0)),
            pl.BlockSpec((D_MODEL, 1), lambda i: (0, 0)),
            pl.BlockSpec(memory_space=pltpu.SMEM),
        ],
        out_specs=[
            pl.BlockSpec((BQ, S), lambda i: (i, 0)),
            pl.BlockSpec((BQ, 1), lambda i: (i, 0)),
        ],
        out_shape=[
            jax.ShapeDtypeStruct((S, S), jnp.float32),
            jax.ShapeDtypeStruct((S, 1), jnp.float32),
        ],
    )(qi, ki, qb, kb, iw, pooled, cw, cb)


def _attn_body(q_ref, k_ref, v_ref, scores_ref, thr_ref, out_ref):
    add_mask = jnp.where(scores_ref[...] >= thr_ref[...], 0.0, NEG_INF)
    scale = 1.0 / (HEAD_DIM ** 0.5)
    logits = _dot(q_ref[...], k_ref[...], ((1,), (1,))) * scale
    logits = logits + add_mask
    m = jnp.max(logits, axis=1, keepdims=True)
    p = jnp.exp(logits - m)
    p = p / jnp.sum(p, axis=1, keepdims=True)
    out_ref[...] = _dot(p, v_ref[...], ((1,), (0,)))


def _run_attn(q, k, v, scores, thr):
    return pl.pallas_call(
        _attn_body,
        grid=(N_QBLK, N_HEADS),
        in_specs=[
            pl.BlockSpec((BQ, HEAD_DIM), lambda i, hh: (i, hh)),
            pl.BlockSpec((S, HEAD_DIM), lambda i, hh: (0, hh)),
            pl.BlockSpec((S, HEAD_DIM), lambda i, hh: (0, hh)),
            pl.BlockSpec((BQ, S), lambda i, hh: (i, 0)),
            pl.BlockSpec((BQ, 1), lambda i, hh: (i, 0)),
        ],
        out_specs=pl.BlockSpec((BQ, HEAD_DIM), lambda i, hh: (i, hh)),
        out_shape=jax.ShapeDtypeStruct((S, D_MODEL), jnp.float32),
    )(q, k, v, scores, thr)


def _oproj_body(x_ref, w_ref, out_ref):
    out_ref[...] = _dot(x_ref[...], w_ref[...], ((1,), (0,)))


def _run_oproj(x, w):
    return pl.pallas_call(
        _oproj_body,
        grid=(N_QBLK,),
        in_specs=[
            pl.BlockSpec((BQ, D_MODEL), lambda i: (i, 0)),
            pl.BlockSpec((D_MODEL, D_MODEL), lambda i: (0, 0)),
        ],
        out_specs=pl.BlockSpec((BQ, D_MODEL), lambda i: (i, 0)),
        out_shape=jax.ShapeDtypeStruct((S, D_MODEL), jnp.float32),
    )(x, w)


def kernel(hidden_states, q_idx_W, q_idx_b, k_idx_W, k_idx_b, idx_weights,
           ctrl_W, ctrl_b, Wq, Wk, Wv, Wo):
    h = hidden_states[0]

    w_cat = jnp.concatenate([q_idx_W, k_idx_W, Wq, Wk, Wv], axis=1)
    proj, pooled = _run_proj(h, w_cat)

    nidx = IDX_HEADS * IDX_DIM
    qi = proj[:, :nidx]
    ki = proj[:, nidx:2 * nidx]
    q = proj[:, 2 * nidx:2 * nidx + D_MODEL]
    k = proj[:, 2 * nidx + D_MODEL:2 * nidx + 2 * D_MODEL]
    v = proj[:, 2 * nidx + 2 * D_MODEL:]

    scores, thr = _run_scores(qi, ki, q_idx_b[None, :], k_idx_b[None, :],
                              idx_weights, pooled, ctrl_W, ctrl_b)

    attn = _run_attn(q, k, v, scores, thr)
    out = _run_oproj(attn, Wo)
    return out[None]


# speedup vs baseline: 2.7249x; 1.0048x over previous
import functools

import jax
import jax.numpy as jnp
from jax import lax
from jax.experimental import pallas as pl
from jax.experimental.pallas import tpu as pltpu

D_MODEL = 2048
N_HEADS = 16
HEAD_DIM = D_MODEL // N_HEADS
IDX_HEADS = 4
IDX_DIM = 64
S = 2048
NEG_INF = -1e9

BQ = 256
N_QBLK = S // BQ
WBLK = 256

def _dot(a, b, dims):
    return lax.dot_general(a.astype(jnp.bfloat16), b.astype(jnp.bfloat16),
                           dimension_numbers=(dims, ((), ())),
                           preferred_element_type=jnp.float32)


def _proj_body(h_ref, w_ref, out_ref, pooled_ref):
    i = pl.program_id(0)
    j = pl.program_id(1)
    out_ref[...] = _dot(h_ref[...], w_ref[...], ((1,), (0,)))

    @pl.when(j == 0)
    def _():
        part = jnp.sum(h_ref[...], axis=0, keepdims=True) * (1.0 / S)

        @pl.when(i == 0)
        def _():
            pooled_ref[...] = part

        @pl.when(i > 0)
        def _():
            pooled_ref[...] += part


def _run_proj(h, w_cat):
    bm = 512
    n_wblk = w_cat.shape[1] // WBLK
    return pl.pallas_call(
        _proj_body,
        grid=(S // bm, n_wblk),
        in_specs=[
            pl.BlockSpec((bm, D_MODEL), lambda i, j: (i, 0)),
            pl.BlockSpec((D_MODEL, WBLK), lambda i, j: (0, j)),
        ],
        out_specs=[
            pl.BlockSpec((bm, WBLK), lambda i, j: (i, j)),
            pl.BlockSpec((1, D_MODEL), lambda i, j: (0, 0)),
        ],
        out_shape=[
            jax.ShapeDtypeStruct((S, w_cat.shape[1]), jnp.float32),
            jax.ShapeDtypeStruct((1, D_MODEL), jnp.float32),
        ],
    )(h, w_cat)


def _f32_key(x):
    u = lax.bitcast_convert_type(x, jnp.uint32)
    neg = (u >> 31) == jnp.uint32(1)
    return jnp.where(neg, ~u, u | jnp.uint32(0x80000000))


def _key_to_f32(key):
    neg = (key >> 31) == jnp.uint32(0)
    u = jnp.where(neg, ~key, key & jnp.uint32(0x7FFFFFFF))
    return lax.bitcast_convert_type(u, jnp.float32)


def _scores_body(qi_ref, ki_ref, qb_ref, kb_ref, iw_ref, pooled_ref, cw_ref,
                 cb_ref, mask_ref):
    acc = jnp.zeros((BQ, S), jnp.float32)
    for hh in range(IDX_HEADS):
        sl = slice(hh * IDX_DIM, (hh + 1) * IDX_DIM)
        q = qi_ref[:, sl] + qb_ref[:, sl]
        k = ki_ref[:, sl] + kb_ref[:, sl]
        dp = _dot(q, k, ((1,), (1,)))
        acc = acc + iw_ref[hh] * jnp.maximum(dp, 0.0)

    r = _dot(pooled_ref[...], cw_ref[...], ((1,), (0,)))[0, 0] + cb_ref[0]
    ratio = 1.0 / (1.0 + jnp.exp(-r))
    kf = jnp.clip(lax.round(ratio * S, lax.RoundingMethod.TO_NEAREST_EVEN),
                  1.0, float(S))
    kint = kf.astype(jnp.int32)

    keys = _f32_key(acc)

    def step(i, cur):
        bit = jnp.uint32(1) << (jnp.uint32(31) - i.astype(jnp.uint32))
        cand = cur | bit
        cnt = jnp.sum((keys >= cand).astype(jnp.int32), axis=1, keepdims=True)
        return jnp.where(cnt >= kint, cand, cur)

    cur = lax.fori_loop(0, 32, step, jnp.zeros((BQ, 1), jnp.uint32))
    mask_ref[...] = jnp.where(keys >= cur, 0.0, NEG_INF)


def _run_scores(qi, ki, qb, kb, iw, pooled, cw, cb):
    return pl.pallas_call(
        _scores_body,
        grid=(N_QBLK,),
        in_specs=[
            pl.BlockSpec((BQ, IDX_HEADS * IDX_DIM), lambda i: (i, 0)),
            pl.BlockSpec((S, IDX_HEADS * IDX_DIM), lambda i: (0, 0)),
            pl.BlockSpec((1, IDX_HEADS * IDX_DIM), lambda i: (0, 0)),
            pl.BlockSpec((1, IDX_HEADS * IDX_DIM), lambda i: (0, 0)),
            pl.BlockSpec(memory_space=pltpu.SMEM),
            pl.BlockSpec((1, D_MODEL), lambda i: (0, 0)),
            pl.BlockSpec((D_MODEL, 1), lambda i: (0, 0)),
            pl.BlockSpec(memory_space=pltpu.SMEM),
        ],
        out_specs=pl.BlockSpec((BQ, S), lambda i: (i, 0)),
        out_shape=jax.ShapeDtypeStruct((S, S), jnp.float32),
    )(qi, ki, qb, kb, iw, pooled, cw, cb)


def _attn_body(q_ref, k_ref, v_ref, mask_ref, out_ref):
    scale = 1.0 / (HEAD_DIM ** 0.5)
    logits = _dot(q_ref[...], k_ref[...], ((1,), (1,))) * scale
    logits = logits + mask_ref[...]
    m = jnp.max(logits, axis=1, keepdims=True)
    p = jnp.exp(logits - m)
    p = p / jnp.sum(p, axis=1, keepdims=True)
    out_ref[...] = _dot(p, v_ref[...], ((1,), (0,)))


def _run_attn(q, k, v, mask):
    return pl.pallas_call(
        _attn_body,
        grid=(N_QBLK, N_HEADS),
        in_specs=[
            pl.BlockSpec((BQ, HEAD_DIM), lambda i, hh: (i, hh)),
            pl.BlockSpec((S, HEAD_DIM), lambda i, hh: (0, hh)),
            pl.BlockSpec((S, HEAD_DIM), lambda i, hh: (0, hh)),
            pl.BlockSpec((BQ, S), lambda i, hh: (i, 0)),
        ],
        out_specs=pl.BlockSpec((BQ, HEAD_DIM), lambda i, hh: (i, hh)),
        out_shape=jax.ShapeDtypeStruct((S, D_MODEL), jnp.float32),
    )(q, k, v, mask)


def _oproj_body(x_ref, w_ref, out_ref):
    out_ref[...] = _dot(x_ref[...], w_ref[...], ((1,), (0,)))


def _run_oproj(x, w):
    return pl.pallas_call(
        _oproj_body,
        grid=(N_QBLK,),
        in_specs=[
            pl.BlockSpec((BQ, D_MODEL), lambda i: (i, 0)),
            pl.BlockSpec((D_MODEL, D_MODEL), lambda i: (0, 0)),
        ],
        out_specs=pl.BlockSpec((BQ, D_MODEL), lambda i: (i, 0)),
        out_shape=jax.ShapeDtypeStruct((S, D_MODEL), jnp.float32),
    )(x, w)


def kernel(hidden_states, q_idx_W, q_idx_b, k_idx_W, k_idx_b, idx_weights,
           ctrl_W, ctrl_b, Wq, Wk, Wv, Wo):
    h = hidden_states[0]

    w_cat = jnp.concatenate([q_idx_W, k_idx_W, Wq, Wk, Wv], axis=1)
    proj, pooled = _run_proj(h, w_cat)

    nidx = IDX_HEADS * IDX_DIM
    qi = proj[:, :nidx]
    ki = proj[:, nidx:2 * nidx]
    q = proj[:, 2 * nidx:2 * nidx + D_MODEL]
    k = proj[:, 2 * nidx + D_MODEL:2 * nidx + 2 * D_MODEL]
    v = proj[:, 2 * nidx + 2 * D_MODEL:]

    mask = _run_scores(qi, ki, q_idx_b[None, :], k_idx_b[None, :],
                       idx_weights, pooled, ctrl_W, ctrl_b)

    attn = _run_attn(q, k, v, mask)
    out = _run_oproj(attn, Wo)
    return out[None]


# speedup vs baseline: 3.1771x; 1.1660x over previous
import jax
import jax.numpy as jnp
from jax import lax
from jax.experimental import pallas as pl
from jax.experimental.pallas import tpu as pltpu

D_MODEL = 2048
N_HEADS = 16
HEAD_DIM = D_MODEL // N_HEADS
IDX_HEADS = 4
IDX_DIM = 64
NIDX = IDX_HEADS * IDX_DIM
S = 2048
NEG_INF = -1e9

BQ = 256
N_QBLK = S // BQ
WBLK = 512


def _dot(a, b, dims):
    return lax.dot_general(a.astype(jnp.bfloat16), b.astype(jnp.bfloat16),
                           dimension_numbers=(dims, ((), ())),
                           preferred_element_type=jnp.float32)


def _proj_idx_body(h_ref, w_ref, out_ref, pooled_ref):
    i = pl.program_id(0)
    out_ref[...] = _dot(h_ref[...], w_ref[...], ((1,), (0,)))

    part = jnp.sum(h_ref[...].astype(jnp.float32), axis=0,
                   keepdims=True) * (1.0 / S)

    @pl.when(i == 0)
    def _():
        pooled_ref[...] = part

    @pl.when(i > 0)
    def _():
        pooled_ref[...] += part


def _run_proj_idx(h, w_idx):
    bm = 512
    return pl.pallas_call(
        _proj_idx_body,
        grid=(S // bm,),
        in_specs=[
            pl.BlockSpec((bm, D_MODEL), lambda i: (i, 0)),
            pl.BlockSpec((D_MODEL, 2 * NIDX), lambda i: (0, 0)),
        ],
        out_specs=[
            pl.BlockSpec((bm, 2 * NIDX), lambda i: (i, 0)),
            pl.BlockSpec((1, D_MODEL), lambda i: (0, 0)),
        ],
        out_shape=[
            jax.ShapeDtypeStruct((S, 2 * NIDX), jnp.float32),
            jax.ShapeDtypeStruct((1, D_MODEL), jnp.float32),
        ],
    )(h, w_idx)


def _proj_qkv_body(h_ref, w_ref, out_ref):
    out_ref[...] = _dot(h_ref[...], w_ref[...],
                        ((1,), (0,))).astype(jnp.bfloat16)


def _run_proj_qkv(h, w_qkv):
    bm = 512
    n_wblk = w_qkv.shape[1] // WBLK
    return pl.pallas_call(
        _proj_qkv_body,
        grid=(S // bm, n_wblk),
        in_specs=[
            pl.BlockSpec((bm, D_MODEL), lambda i, j: (i, 0)),
            pl.BlockSpec((D_MODEL, WBLK), lambda i, j: (0, j)),
        ],
        out_specs=pl.BlockSpec((bm, WBLK), lambda i, j: (i, j)),
        out_shape=jax.ShapeDtypeStruct((S, w_qkv.shape[1]), jnp.bfloat16),
    )(h, w_qkv)


def _f32_key(x):
    u = lax.bitcast_convert_type(x, jnp.uint32)
    neg = (u >> 31) == jnp.uint32(1)
    return jnp.where(neg, ~u, u | jnp.uint32(0x80000000))


def _scores_body(qi_ref, ki_ref, qb_ref, kb_ref, iw_ref, pooled_ref, cw_ref,
                 cb_ref, mask_ref):
    acc = jnp.zeros((BQ, S), jnp.float32)
    for hh in range(IDX_HEADS):
        sl = slice(hh * IDX_DIM, (hh + 1) * IDX_DIM)
        q = qi_ref[:, sl] + qb_ref[:, sl]
        k = ki_ref[:, sl] + kb_ref[:, sl]
        dp = _dot(q, k, ((1,), (1,)))
        acc = acc + iw_ref[hh] * jnp.maximum(dp, 0.0)

    r = _dot(pooled_ref[...], cw_ref[...], ((1,), (0,)))[0, 0] + cb_ref[0]
    ratio = 1.0 / (1.0 + jnp.exp(-r))
    kf = jnp.clip(lax.round(ratio * S, lax.RoundingMethod.TO_NEAREST_EVEN),
                  1.0, float(S))
    kint = kf.astype(jnp.int32)

    keys = _f32_key(acc)

    def step(i, cur):
        bit = jnp.uint32(1) << (jnp.uint32(31) - i.astype(jnp.uint32))
        cand = cur | bit
        cnt = jnp.sum((keys >= cand).astype(jnp.int32), axis=1, keepdims=True)
        return jnp.where(cnt >= kint, cand, cur)

    cur = lax.fori_loop(0, 32, step, jnp.zeros((BQ, 1), jnp.uint32))
    mask_ref[...] = jnp.where(keys >= cur, 0.0, NEG_INF)


def _run_scores(qi, ki, qb, kb, iw, pooled, cw, cb):
    return pl.pallas_call(
        _scores_body,
        grid=(N_QBLK,),
        in_specs=[
            pl.BlockSpec((BQ, NIDX), lambda i: (i, 0)),
            pl.BlockSpec((S, NIDX), lambda i: (0, 0)),
            pl.BlockSpec((1, NIDX), lambda i: (0, 0)),
            pl.BlockSpec((1, NIDX), lambda i: (0, 0)),
            pl.BlockSpec(memory_space=pltpu.SMEM),
            pl.BlockSpec((1, D_MODEL), lambda i: (0, 0)),
            pl.BlockSpec((D_MODEL, 1), lambda i: (0, 0)),
            pl.BlockSpec(memory_space=pltpu.SMEM),
        ],
        out_specs=pl.BlockSpec((BQ, S), lambda i: (i, 0)),
        out_shape=jax.ShapeDtypeStruct((S, S), jnp.float32),
    )(qi, ki, qb, kb, iw, pooled, cw, cb)


def _attn_body(q_ref, k_ref, v_ref, mask_ref, out_ref):
    scale = 1.0 / (HEAD_DIM ** 0.5)
    logits = lax.dot_general(q_ref[...], k_ref[...],
                             dimension_numbers=((((1,), (1,))), ((), ())),
                             preferred_element_type=jnp.float32) * scale
    logits = logits + mask_ref[...]
    m = jnp.max(logits, axis=1, keepdims=True)
    p = jnp.exp(logits - m)
    z = jnp.sum(p, axis=1, keepdims=True)
    pv = lax.dot_general(p.astype(jnp.bfloat16), v_ref[...],
                         dimension_numbers=((((1,), (0,))), ((), ())),
                         preferred_element_type=jnp.float32)
    out_ref[...] = (pv * (1.0 / z)).astype(jnp.bfloat16)


def _run_attn(q, k, v, mask):
    return pl.pallas_call(
        _attn_body,
        grid=(N_QBLK, N_HEADS),
        in_specs=[
            pl.BlockSpec((BQ, HEAD_DIM), lambda i, hh: (i, hh)),
            pl.BlockSpec((S, HEAD_DIM), lambda i, hh: (0, hh)),
            pl.BlockSpec((S, HEAD_DIM), lambda i, hh: (0, hh)),
            pl.BlockSpec((BQ, S), lambda i, hh: (i, 0)),
        ],
        out_specs=pl.BlockSpec((BQ, HEAD_DIM), lambda i, hh: (i, hh)),
        out_shape=jax.ShapeDtypeStruct((S, D_MODEL), jnp.bfloat16),
    )(q, k, v, mask)


def _oproj_body(x_ref, w_ref, out_ref):
    out_ref[...] = lax.dot_general(x_ref[...], w_ref[...],
                                   dimension_numbers=((((1,), (0,))), ((), ())),
                                   preferred_element_type=jnp.float32)


def _run_oproj(x, w):
    return pl.pallas_call(
        _oproj_body,
        grid=(N_QBLK,),
        in_specs=[
            pl.BlockSpec((BQ, D_MODEL), lambda i: (i, 0)),
            pl.BlockSpec((D_MODEL, D_MODEL), lambda i: (0, 0)),
        ],
        out_specs=pl.BlockSpec((BQ, D_MODEL), lambda i: (i, 0)),
        out_shape=jax.ShapeDtypeStruct((S, D_MODEL), jnp.float32),
    )(x, w)


def kernel(hidden_states, q_idx_W, q_idx_b, k_idx_W, k_idx_b, idx_weights,
           ctrl_W, ctrl_b, Wq, Wk, Wv, Wo):
    h16 = hidden_states[0].astype(jnp.bfloat16)

    w_idx = jnp.concatenate([q_idx_W, k_idx_W], axis=1).astype(jnp.bfloat16)
    w_qkv = jnp.concatenate([Wq, Wk, Wv], axis=1).astype(jnp.bfloat16)

    proj_idx, pooled = _run_proj_idx(h16, w_idx)
    qkv = _run_proj_qkv(h16, w_qkv)

    qi = proj_idx[:, :NIDX]
    ki = proj_idx[:, NIDX:]
    q = qkv[:, :D_MODEL]
    k = qkv[:, D_MODEL:2 * D_MODEL]
    v = qkv[:, 2 * D_MODEL:]

    mask = _run_scores(qi, ki, q_idx_b[None, :], k_idx_b[None, :],
                       idx_weights, pooled, ctrl_W, ctrl_b)

    attn = _run_attn(q, k, v, mask)
    out = _run_oproj(attn, Wo.astype(jnp.bfloat16))
    return out[None]
